# baseline (speedup 1.0000x reference)
import functools

import jax
import jax.numpy as jnp
from jax import lax
from jax.experimental import pallas as pl
from jax.experimental.pallas import tpu as pltpu
from jax.experimental.pallas import tpu_sc as plsc

OUT_H = 7
OUT_W = 7
SCALE = 0.25
GRID = 2
NSAMP = OUT_H * OUT_W * GRID * GRID * 4


def _index_body(H, W, rois_ref, idx_ref, w_ref):
    rois = rois_ref[...]
    batch = rois[:, 0:1].astype(jnp.int32)
    sx1 = rois[:, 1:2] * SCALE
    sy1 = rois[:, 2:3] * SCALE
    sx2 = rois[:, 3:4] * SCALE
    sy2 = rois[:, 4:5] * SCALE
    roi_w = jnp.maximum(sx2 - sx1, 1.0)
    roi_h = jnp.maximum(sy2 - sy1, 1.0)
    bin_h = roi_h / OUT_H
    bin_w = roi_w / OUT_W

    j = lax.broadcasted_iota(jnp.int32, (1, NSAMP), 1)
    bin_i = j // 16
    e = j % 16
    ph = (bin_i // OUT_W).astype(jnp.float32)
    pw = (bin_i % OUT_W).astype(jnp.float32)
    gy = e // 8
    gx = (e // 4) % 2
    a = (e // 2) % 2
    b = e % 2

    gyf = (gy.astype(jnp.float32) + 0.5) / GRID
    gxf = (gx.astype(jnp.float32) + 0.5) / GRID
    ys = sy1 + (ph + gyf) * bin_h
    xs = sx1 + (pw + gxf) * bin_w

    valid = (ys >= -1.0) & (ys <= H) & (xs >= -1.0) & (xs <= W)

    def axis_terms(v, sel_hi, dim):
        v = jnp.maximum(v, 0.0)
        l0 = jnp.floor(v).astype(jnp.int32)
        cond = l0 >= dim - 1
        low = jnp.where(cond, dim - 1, l0)
        high = jnp.where(cond, dim - 1, l0 + 1)
        v = jnp.where(cond, jnp.float32(dim - 1), v)
        lv = v - low.astype(jnp.float32)
        wt = jnp.where(sel_hi == 1, lv, 1.0 - lv)
        sel = jnp.where(sel_hi == 1, high, low)
        return sel, wt

    ysel, wy = axis_terms(ys, a, H)
    xsel, wx = axis_terms(xs, b, W)

    idx_ref[...] = batch * (H * W) + ysel * W + xsel
    w_ref[...] = wy * wx * valid.astype(jnp.float32) * 0.25


NBUF = 4
QUAD = 4
PACK_P = 64


def _pack_body(C, NPIX, feat_ref, out_ref, in_v, out_v, isems, osems):
    NC = 2
    wid = lax.axis_index("s") * NC + lax.axis_index("c")
    total = (NPIX + PACK_P - 1) // PACK_P
    per_tile = (total + 31) // 32
    cbase = wid * per_tile
    cend = jnp.minimum(cbase + per_tile, total)
    half = C // 2

    def issue_in(q, k):
        pltpu.async_copy(feat_ref.at[pl.ds(q * PACK_P, PACK_P)],
                         in_v[k], isems[k])

    for k in range(2):
        @pl.when(cbase + k < cend)
        def _prime(_k=k):
            issue_in(cbase + _k, _k)

    def body(u, carry):
        for k in range(2):
            q = cbase + u * 2 + k

            @pl.when(q < cend)
            def _work(_k=k, _q=q):
                pltpu.make_async_copy(feat_ref.at[pl.ds(0, PACK_P)],
                                      in_v[_k], isems[_k]).wait()

                @pl.when(u > 0)
                def _drain_out():
                    pltpu.make_async_copy(out_v[_k],
                                          out_ref.at[pl.ds(0, PACK_P)],
                                          osems[_k]).wait()

                def pbody(p, car):
                    for cc in range(half // 16):
                        a = in_v[_k][p, pl.ds(cc * 16, 16)]
                        b = in_v[_k][p, pl.ds(half + cc * 16, 16)]
                        w32 = plsc.pack(a, b,
                                        format=plsc.PackFormat.INTERLEAVED)
                        out_v[_k][p, pl.ds(cc * 16, 16)] = plsc.bitcast(
                            w32, jnp.int32)
                    return car

                lax.fori_loop(0, PACK_P, pbody, 0, unroll=False)
                pltpu.async_copy(out_v[_k],
                                 out_ref.at[pl.ds(_q * PACK_P, PACK_P)],
                                 osems[_k])

                @pl.when(_q + 2 < cend)
                def _issue_next():
                    issue_in(_q + 2, _k)

        return carry

    lax.fori_loop(0, (per_tile + 1) // 2, body, 0, unroll=False)
    for k in range(2):
        @pl.when(cbase + k < cend)
        def _drain(_k=k):
            pltpu.make_async_copy(out_v[_k], out_ref.at[pl.ds(0, PACK_P)],
                                  osems[_k]).wait()


def _sc_body(C, RPT, feat_ref, idx_ref, w_ref, out_ref,
             idx_v, w_v, rows, stg_v, gsems, ssems):
    NC = 2
    wid = lax.axis_index("s") * NC + lax.axis_index("c")
    chunk = RPT * NSAMP
    pltpu.sync_copy(idx_ref.at[pl.ds(wid * chunk, chunk)], idx_v)
    pltpu.sync_copy(w_ref.at[pl.ds(wid * chunk, chunk)], w_v)

    nbin = OUT_H * OUT_W
    total = RPT * nbin
    tq = total // QUAD
    rbase = wid * RPT
    roi_words = C * nbin
    half = C // 2
    nblk = C // 32

    iota49 = lax.iota(jnp.int32, 16) * nbin
    gdims = lax.GatherDimensionNumbers(
        offset_dims=(), collapsed_slice_dims=(0,), start_index_map=(0,))

    def issue_gather(q, rb, sem):
        pltpu.async_copy(
            feat_ref.at[idx_v.at[pl.ds(q * (QUAD * 16), QUAD * 16)]], rb, sem)

    for k in range(NBUF):
        issue_gather(k, rows[k], gsems[k])

    def bin_compute(rb, j, t):
        bin_ = lax.rem(t, nbin)
        roi = lax.div(t, nbin)
        parity = lax.rem(roi, 2)
        pbase = parity * roi_words

        first = jnp.logical_and(bin_ == 0, roi >= 2)

        @pl.when(jnp.logical_and(first, parity == 0))
        def _wait_stg0():
            pltpu.make_async_copy(stg_v.at[pl.ds(0, roi_words)],
                                  out_ref.at[pl.ds(0, roi_words)],
                                  ssems[0]).wait()

        @pl.when(jnp.logical_and(first, parity == 1))
        def _wait_stg1():
            pltpu.make_async_copy(stg_v.at[pl.ds(0, roi_words)],
                                  out_ref.at[pl.ds(0, roi_words)],
                                  ssems[1]).wait()

        w_vec = w_v[pl.ds(t * 16, 16)]
        wp = []
        for i in range(16):
            wb = lax.gather(w_vec, jnp.full((16, 1), i, jnp.int32), gdims, (1,),
                            mode=lax.GatherScatterMode.PROMISE_IN_BOUNDS)
            wp.append(plsc.pack(wb, wb, format=plsc.PackFormat.INTERLEAVED))
        for cc in range(nblk):
            terms = [
                wp[i] * plsc.bitcast(rb[j * 16 + i, pl.ds(cc * 16, 16)],
                                     jnp.bfloat16)
                for i in range(16)
            ]
            while len(terms) > 1:
                terms = [terms[m] + terms[m + 1]
                         for m in range(0, len(terms), 2)]
            a, b = plsc.unpack(terms[0], format=plsc.PackFormat.INTERLEAVED)
            scal = pbase + cc * (16 * nbin) + bin_
            idxa = iota49 + scal
            plsc.store_scatter(stg_v, [idxa], a)
            plsc.store_scatter(stg_v, [idxa + (C // 2) * nbin], b)

        last = bin_ == nbin - 1
        g = (rbase + roi) * roi_words

        @pl.when(jnp.logical_and(last, parity == 0))
        def _store0():
            pltpu.async_copy(stg_v.at[pl.ds(0, roi_words)],
                             out_ref.at[pl.ds(g, roi_words)], ssems[0])

        @pl.when(jnp.logical_and(last, parity == 1))
        def _store1():
            pltpu.async_copy(stg_v.at[pl.ds(roi_words, roi_words)],
                             out_ref.at[pl.ds(g, roi_words)], ssems[1])

    def body(u, carry):
        for k in range(NBUF):
            q = u * NBUF + k
            pltpu.make_async_copy(feat_ref.at[pl.ds(0, QUAD * 16)],
                                  rows[k], gsems[k]).wait()

            def jbody(j, car, _k=k, _q=q):
                bin_compute(rows[_k], j, _q * QUAD + j)
                return car

            lax.fori_loop(0, QUAD, jbody, 0, unroll=False)

            @pl.when(q + NBUF < tq)
            def _issue_next(_k=k, _q=q):
                issue_gather(_q + NBUF, rows[_k], gsems[_k])

        return carry

    lax.fori_loop(0, tq // NBUF, body, 0, unroll=False)
    pltpu.make_async_copy(stg_v.at[pl.ds(0, roi_words)],
                          out_ref.at[pl.ds(0, roi_words)], ssems[0]).wait()
    pltpu.make_async_copy(stg_v.at[pl.ds(0, roi_words)],
                          out_ref.at[pl.ds(0, roi_words)], ssems[1]).wait()


def kernel(features, rois):
    N, C, H, W = features.shape
    R = rois.shape[0]
    NW = 32
    RP = ((R + NW - 1) // NW) * NW
    RPT = RP // NW
    nbin = OUT_H * OUT_W

    rois_p = jnp.pad(rois, ((0, RP - R), (0, 0)))
    ftab = jnp.transpose(features, (0, 2, 3, 1)).reshape(N * H * W, C)
    mesh = plsc.VectorSubcoreMesh(core_axis_name="c", subcore_axis_name="s")
    table = pl.kernel(
        functools.partial(_pack_body, C, N * H * W),
        out_type=jax.ShapeDtypeStruct((N * H * W, C // 2), jnp.int32),
        mesh=mesh,
        compiler_params=pltpu.CompilerParams(needs_layout_passes=False),
        scratch_types=[
            [pltpu.VMEM((PACK_P, C), jnp.float32) for _ in range(2)],
            [pltpu.VMEM((PACK_P, C // 2), jnp.int32) for _ in range(2)],
            [pltpu.SemaphoreType.DMA for _ in range(2)],
            [pltpu.SemaphoreType.DMA for _ in range(2)],
        ],
    )(ftab)

    idx, wts = pl.pallas_call(
        functools.partial(_index_body, H, W),
        out_shape=[
            jax.ShapeDtypeStruct((RP, NSAMP), jnp.int32),
            jax.ShapeDtypeStruct((RP, NSAMP), jnp.float32),
        ],
    )(rois_p)

    mesh = plsc.VectorSubcoreMesh(core_axis_name="c", subcore_axis_name="s")
    out_flat = pl.kernel(
        functools.partial(_sc_body, C, RPT),
        out_type=jax.ShapeDtypeStruct((RP * nbin * C,), jnp.float32),
        mesh=mesh,
        compiler_params=pltpu.CompilerParams(needs_layout_passes=False),
        scratch_types=[
            pltpu.VMEM((RPT * NSAMP,), jnp.int32),
            pltpu.VMEM((RPT * NSAMP,), jnp.float32),
            [pltpu.VMEM((QUAD * 16, C // 2), jnp.int32) for _ in range(NBUF)],
            pltpu.VMEM((2 * nbin * C,), jnp.float32),
            [pltpu.SemaphoreType.DMA for _ in range(NBUF)],
            [pltpu.SemaphoreType.DMA for _ in range(2)],
        ],
    )(table, idx.reshape(-1), wts.reshape(-1))

    return out_flat.reshape(RP, C, OUT_H, OUT_W)[:R]


# speedup vs baseline: 1.5127x; 1.5127x over previous
import functools

import jax
import jax.numpy as jnp
from jax import lax
from jax.experimental import pallas as pl
from jax.experimental.pallas import tpu as pltpu
from jax.experimental.pallas import tpu_sc as plsc

OUT_H = 7
OUT_W = 7
SCALE = 0.25
GRID = 2
NSAMP = OUT_H * OUT_W * GRID * GRID * 4


def _index_body(H, W, rois_ref, idx_ref, w_ref):
    rois = rois_ref[...]
    batch = rois[:, 0:1].astype(jnp.int32)
    sx1 = rois[:, 1:2] * SCALE
    sy1 = rois[:, 2:3] * SCALE
    sx2 = rois[:, 3:4] * SCALE
    sy2 = rois[:, 4:5] * SCALE
    roi_w = jnp.maximum(sx2 - sx1, 1.0)
    roi_h = jnp.maximum(sy2 - sy1, 1.0)
    bin_h = roi_h / OUT_H
    bin_w = roi_w / OUT_W

    j = lax.broadcasted_iota(jnp.int32, (1, NSAMP), 1)
    bin_i = j // 16
    e = j % 16
    ph = (bin_i // OUT_W).astype(jnp.float32)
    pw = (bin_i % OUT_W).astype(jnp.float32)
    gy = e // 8
    gx = (e // 4) % 2
    a = (e // 2) % 2
    b = e % 2

    gyf = (gy.astype(jnp.float32) + 0.5) / GRID
    gxf = (gx.astype(jnp.float32) + 0.5) / GRID
    ys = sy1 + (ph + gyf) * bin_h
    xs = sx1 + (pw + gxf) * bin_w

    valid = (ys >= -1.0) & (ys <= H) & (xs >= -1.0) & (xs <= W)

    def axis_terms(v, sel_hi, dim):
        v = jnp.maximum(v, 0.0)
        l0 = jnp.floor(v).astype(jnp.int32)
        cond = l0 >= dim - 1
        low = jnp.where(cond, dim - 1, l0)
        high = jnp.where(cond, dim - 1, l0 + 1)
        v = jnp.where(cond, jnp.float32(dim - 1), v)
        lv = v - low.astype(jnp.float32)
        wt = jnp.where(sel_hi == 1, lv, 1.0 - lv)
        sel = jnp.where(sel_hi == 1, high, low)
        return sel, wt

    ysel, wy = axis_terms(ys, a, H)
    xsel, wx = axis_terms(xs, b, W)

    idx_ref[...] = batch * (H * W) + ysel * W + xsel
    w_ref[...] = wy * wx * valid.astype(jnp.float32) * 0.25


NBUF = 4
QUAD = 4
PACK_P = 64


def _pack_body(C, NPIX, feat_ref, out_ref, in_v, out_v, isems, osems):
    NC = 2
    wid = lax.axis_index("s") * NC + lax.axis_index("c")
    total = (NPIX + PACK_P - 1) // PACK_P
    per_tile = (total + 31) // 32
    cbase = wid * per_tile
    cend = jnp.minimum(cbase + per_tile, total)
    half = C // 2

    def issue_in(q, k):
        pltpu.async_copy(feat_ref.at[pl.ds(q * PACK_P, PACK_P)],
                         in_v[k], isems[k])

    for k in range(2):
        @pl.when(cbase + k < cend)
        def _prime(_k=k):
            issue_in(cbase + _k, _k)

    def body(u, carry):
        for k in range(2):
            q = cbase + u * 2 + k

            @pl.when(q < cend)
            def _work(_k=k, _q=q):
                pltpu.make_async_copy(feat_ref.at[pl.ds(0, PACK_P)],
                                      in_v[_k], isems[_k]).wait()

                @pl.when(u > 0)
                def _drain_out():
                    pltpu.make_async_copy(out_v[_k],
                                          out_ref.at[pl.ds(0, PACK_P)],
                                          osems[_k]).wait()

                def pbody(p, car):
                    for cc in range(half // 16):
                        a = in_v[_k][p, pl.ds(cc * 16, 16)]
                        b = in_v[_k][p, pl.ds(half + cc * 16, 16)]
                        w32 = plsc.pack(a, b,
                                        format=plsc.PackFormat.INTERLEAVED)
                        out_v[_k][p, pl.ds(cc * 16, 16)] = plsc.bitcast(
                            w32, jnp.int32)
                    return car

                lax.fori_loop(0, PACK_P, pbody, 0, unroll=False)
                pltpu.async_copy(out_v[_k],
                                 out_ref.at[pl.ds(_q * PACK_P, PACK_P)],
                                 osems[_k])

                @pl.when(_q + 2 < cend)
                def _issue_next():
                    issue_in(_q + 2, _k)

        return carry

    lax.fori_loop(0, (per_tile + 1) // 2, body, 0, unroll=False)
    for k in range(2):
        @pl.when(cbase + k < cend)
        def _drain(_k=k):
            pltpu.make_async_copy(out_v[_k], out_ref.at[pl.ds(0, PACK_P)],
                                  osems[_k]).wait()


def _sc_body(C, RPT, feat_ref, idx_ref, w_ref, out_ref,
             idx_v, w_v, rows, stg, gsems, ssems):
    NC = 2
    wid = lax.axis_index("s") * NC + lax.axis_index("c")
    chunk = RPT * NSAMP
    pltpu.sync_copy(idx_ref.at[pl.ds(wid * chunk, chunk)], idx_v)
    pltpu.sync_copy(w_ref.at[pl.ds(wid * chunk, chunk)], w_v)

    nbin = OUT_H * OUT_W
    total = RPT * nbin
    tq = total // QUAD
    base = wid * total
    nblk = C // 32

    gdims = lax.GatherDimensionNumbers(
        offset_dims=(), collapsed_slice_dims=(0,), start_index_map=(0,))

    def issue_gather(q, rb, sem):
        pltpu.async_copy(
            feat_ref.at[idx_v.at[pl.ds(q * (QUAD * 16), QUAD * 16)]], rb, sem)

    for k in range(NBUF):
        issue_gather(k, rows[k], gsems[k])

    def bin_compute(rb, sb, j, t):
        w_vec = w_v[pl.ds(t * 16, 16)]
        wp = []
        for i in range(16):
            wb = lax.gather(w_vec, jnp.full((16, 1), i, jnp.int32), gdims, (1,),
                            mode=lax.GatherScatterMode.PROMISE_IN_BOUNDS)
            wp.append(plsc.pack(wb, wb, format=plsc.PackFormat.INTERLEAVED))
        for cc in range(nblk):
            terms = [
                wp[i] * plsc.bitcast(rb[j * 16 + i, pl.ds(cc * 16, 16)],
                                     jnp.bfloat16)
                for i in range(16)
            ]
            while len(terms) > 1:
                terms = [terms[m] + terms[m + 1]
                         for m in range(0, len(terms), 2)]
            a, b = plsc.unpack(terms[0], format=plsc.PackFormat.INTERLEAVED)
            sb[pl.ds(cc * 16, 16)] = a
            sb[pl.ds(C // 2 + cc * 16, 16)] = b

    def body(u, carry):
        for k in range(NBUF):
            q = u * NBUF + k
            pltpu.make_async_copy(feat_ref.at[pl.ds(0, QUAD * 16)],
                                  rows[k], gsems[k]).wait()

            def jbody(j, car, _k=k, _q=q):
                t = _q * QUAD + j

                @pl.when(jnp.logical_or(u > 0, j > 0))
                def _drain_store():
                    pltpu.make_async_copy(stg[_k], out_ref.at[
                        pl.ds(0, C)], ssems[_k]).wait()

                bin_compute(rows[_k], stg[_k], j, t)
                pltpu.async_copy(stg[_k],
                                 out_ref.at[pl.ds((base + t) * C, C)],
                                 ssems[_k])
                return car

            lax.fori_loop(0, QUAD, jbody, 0, unroll=False)

            @pl.when(q + NBUF < tq)
            def _issue_next(_k=k, _q=q):
                issue_gather(_q + NBUF, rows[_k], gsems[_k])

        return carry

    lax.fori_loop(0, tq // NBUF, body, 0, unroll=False)
    for k in range(NBUF):
        pltpu.make_async_copy(stg[k], out_ref.at[pl.ds(0, C)],
                              ssems[k]).wait()


def kernel(features, rois):
    N, C, H, W = features.shape
    R = rois.shape[0]
    NW = 32
    RP = ((R + NW - 1) // NW) * NW
    RPT = RP // NW
    nbin = OUT_H * OUT_W

    rois_p = jnp.pad(rois, ((0, RP - R), (0, 0)))
    ftab = jnp.transpose(features, (0, 2, 3, 1)).reshape(N * H * W, C)
    mesh = plsc.VectorSubcoreMesh(core_axis_name="c", subcore_axis_name="s")
    table = pl.kernel(
        functools.partial(_pack_body, C, N * H * W),
        out_type=jax.ShapeDtypeStruct((N * H * W, C // 2), jnp.int32),
        mesh=mesh,
        compiler_params=pltpu.CompilerParams(needs_layout_passes=False),
        scratch_types=[
            [pltpu.VMEM((PACK_P, C), jnp.float32) for _ in range(2)],
            [pltpu.VMEM((PACK_P, C // 2), jnp.int32) for _ in range(2)],
            [pltpu.SemaphoreType.DMA for _ in range(2)],
            [pltpu.SemaphoreType.DMA for _ in range(2)],
        ],
    )(ftab)

    idx, wts = pl.pallas_call(
        functools.partial(_index_body, H, W),
        out_shape=[
            jax.ShapeDtypeStruct((RP, NSAMP), jnp.int32),
            jax.ShapeDtypeStruct((RP, NSAMP), jnp.float32),
        ],
    )(rois_p)

    mesh = plsc.VectorSubcoreMesh(core_axis_name="c", subcore_axis_name="s")
    out_flat = pl.kernel(
        functools.partial(_sc_body, C, RPT),
        out_type=jax.ShapeDtypeStruct((RP * nbin * C,), jnp.float32),
        mesh=mesh,
        compiler_params=pltpu.CompilerParams(needs_layout_passes=False),
        scratch_types=[
            pltpu.VMEM((RPT * NSAMP,), jnp.int32),
            pltpu.VMEM((RPT * NSAMP,), jnp.float32),
            [pltpu.VMEM((QUAD * 16, C // 2), jnp.int32) for _ in range(NBUF)],
            [pltpu.VMEM((C,), jnp.float32) for _ in range(NBUF)],
            [pltpu.SemaphoreType.DMA for _ in range(NBUF)],
            [pltpu.SemaphoreType.DMA for _ in range(NBUF)],
        ],
    )(table, idx.reshape(-1), wts.reshape(-1))

    out = out_flat.reshape(RP, OUT_H, OUT_W, C)[:R]
    return jnp.transpose(out, (0, 3, 1, 2))
